# Initial kernel scaffold; baseline (speedup 1.0000x reference)
import jax
import jax.numpy as jnp
from jax.experimental import pallas as pl


def kernel(pred_logits, target_label, GD_ema, class_ema):
    raise NotImplementedError("write your pallas kernel here")


# speedup vs baseline: 19.7558x; 19.7558x over previous
import jax
import jax.numpy as jnp
from jax.experimental import pallas as pl
from jax.experimental.pallas import tpu as pltpu

_C = 512
_BINS = 10
_TBLK = 2048


def _ghm_body(x_ref, lbl_ref, gd_ref, cls_ref, out_ref):
    x = x_ref[0]
    lbl = lbl_ref[0]
    m = jnp.max(x, axis=0, keepdims=True)
    s = jnp.sum(jnp.exp(x - m), axis=0, keepdims=True)
    lse = m + jnp.log(s)

    cids = jax.lax.broadcasted_iota(jnp.int32, (_C, _TBLK), 0)
    hit = cids == lbl
    xl = jnp.sum(jnp.where(hit, x, 0.0), axis=0, keepdims=True)
    cw = jnp.sum(jnp.where(hit, cls_ref[...], 0.0), axis=0, keepdims=True)

    p = jnp.exp(xl - lse)
    gd = jnp.abs(p - 1.0)
    idx = jnp.clip(jnp.floor(gd * _BINS).astype(jnp.int32), 0, _BINS - 1)
    gw = jnp.zeros_like(lse)
    for k in range(_BINS):
        gw = jnp.where(idx == k, gd_ref[0, k], gw)

    w = jnp.clip(jnp.sqrt(cw * gw), 1e-10, None)
    out_ref[0, 0] = jnp.sum((lse - xl) / w, axis=1, keepdims=True)


def kernel(pred_logits, target_label, GD_ema, class_ema):
    B, C, T = pred_logits.shape
    nT = T // _TBLK
    lbl3 = target_label.reshape(B, 1, T)
    gd2 = GD_ema.reshape(1, _BINS)
    cls2 = class_ema.reshape(C, 1)
    parts = pl.pallas_call(
        _ghm_body,
        grid=(B, nT),
        in_specs=[
            pl.BlockSpec((1, C, _TBLK), lambda b, t: (b, 0, t)),
            pl.BlockSpec((1, 1, _TBLK), lambda b, t: (b, 0, t)),
            pl.BlockSpec(memory_space=pltpu.SMEM),
            pl.BlockSpec((C, 1), lambda b, t: (0, 0)),
        ],
        out_specs=pl.BlockSpec((1, 1, 1, 1), lambda b, t: (b, t, 0, 0)),
        out_shape=jax.ShapeDtypeStruct((B, nT, 1, 1), jnp.float32),
        compiler_params=pltpu.CompilerParams(
            dimension_semantics=("parallel", "parallel"),
        ),
    )(pred_logits, lbl3, gd2, cls2)
    return jnp.sum(parts) / (B * T)
